# baseline (speedup 1.0000x reference)
import functools

import jax
import jax.numpy as jnp
from jax import lax
from jax.experimental import pallas as pl
from jax.experimental.pallas import tpu as pltpu
from jax.experimental.pallas import tpu_sc as plsc

B = 256
D = 512
SIG = 1536
TOPK = 32
N = 100000
C = 3584
NSTEP = 28
NPAD = NSTEP * C
G = 128
PG = C // G
NG = NSTEP * PG
NW = 32
RPW = B // NW
BIGI = 2**30


def _l2norm(x, eps=1e-12):
    n = jnp.sqrt(jnp.sum(x * x, axis=-1, keepdims=True))
    return x / jnp.maximum(n, eps)


def _pair_signature(emb_a, emb_b):
    a = _l2norm(emb_a)
    b = _l2norm(emb_b)
    sig = jnp.concatenate([a + b, jnp.abs(a - b), a * b], axis=-1)
    return _l2norm(sig)


CH = C // 2


def _k1_body(qb_ref, galA_ref, galB_ref, sc_ref, gm_ref):
    i = pl.program_id(0)
    q = qb_ref[...]
    for h, gref in ((0, galA_ref), (1, galB_ref)):
        x = gref[...]
        s2 = jnp.sum(x * x, axis=1, keepdims=True)
        n = jnp.maximum(jnp.sqrt(s2), 1e-12)
        gb = (x / n).astype(jnp.bfloat16)
        sc = lax.dot_general(q, gb, (((1,), (1,)), ((), ())),
                             preferred_element_type=jnp.float32)
        valid = jnp.int32(N) - i * jnp.int32(C) - h * jnp.int32(CH)
        lane = lax.broadcasted_iota(jnp.int32, (B, CH), 1)
        sc = jnp.where(lane < valid, sc, -jnp.inf)
        sc_ref[:, h * CH:(h + 1) * CH] = sc
        gm_ref[0, :, h * (PG // 2):(h + 1) * (PG // 2)] = (
            jnp.max(sc.reshape(B, PG // 2, G), axis=2))


def _k1(q_bf, gallery):
    return pl.pallas_call(
        _k1_body,
        grid=(NSTEP,),
        in_specs=[
            pl.BlockSpec((B, SIG), lambda i: (0, 0)),
            pl.BlockSpec((CH, SIG), lambda i: (2 * i, 0)),
            pl.BlockSpec((CH, SIG), lambda i: (2 * i + 1, 0)),
        ],
        out_specs=[
            pl.BlockSpec((B, C), lambda i: (0, i)),
            pl.BlockSpec((1, B, PG), lambda i: (i, 0, 0)),
        ],
        out_shape=[
            jax.ShapeDtypeStruct((B, NPAD), jnp.float32),
            jax.ShapeDtypeStruct((NSTEP, B, PG), jnp.float32),
        ],
    )(q_bf, gallery, gallery)


def _k15_body(gm_ref, gsel_ref):
    M = jnp.concatenate([gm_ref[j] for j in range(NSTEP)], axis=1)
    gid = lax.broadcasted_iota(jnp.int32, (B, NG), 1)
    kcol = lax.broadcasted_iota(jnp.int32, (B, TOPK), 1)

    def sel_step(k, carry):
        M, acc = carry
        m = jnp.max(M, axis=1, keepdims=True)
        sel = jnp.min(jnp.where(M == m, gid, BIGI), axis=1, keepdims=True)
        acc = jnp.where(kcol == k, sel, acc)
        M = jnp.where(gid == sel, -jnp.inf, M)
        return M, acc

    _, S = lax.fori_loop(0, TOPK, sel_step,
                         (M, jnp.zeros((B, TOPK), jnp.int32)))
    gsel_ref[...] = S


def _k15(gmax):
    return pl.pallas_call(
        _k15_body,
        in_specs=[pl.BlockSpec((NSTEP, B, PG), lambda: (0, 0, 0))],
        out_specs=pl.BlockSpec((B, TOPK), lambda: (0, 0)),
        out_shape=jax.ShapeDtypeStruct((B, TOPK), jnp.int32),
    )(gmax)


def _k2(scores2, gsel):
    mesh = plsc.VectorSubcoreMesh(core_axis_name="c", subcore_axis_name="s")

    @functools.partial(
        pl.kernel,
        mesh=mesh,
        out_type=jax.ShapeDtypeStruct((B * TOPK, G), jnp.float32),
        scratch_types=[
            pltpu.VMEM((RPW * TOPK,), jnp.int32),
            pltpu.VMEM((RPW * TOPK, G), jnp.float32),
            pltpu.SemaphoreType.DMA,
        ],
    )
    def k2(scores2_hbm, gsel2_hbm, cand_hbm, idx_v, rows_v, sem):
        wid = lax.axis_index("s") * 2 + lax.axis_index("c")
        r0 = wid * RPW
        pltpu.sync_copy(gsel2_hbm.at[pl.ds(r0 * TOPK, RPW * TOPK)], idx_v)
        for j in range(RPW):
            base = (r0 + j) * jnp.int32(NG)
            for h in (0, 16):
                p = j * TOPK + h
                idx_v[pl.ds(p, 16)] = idx_v[pl.ds(p, 16)] + base
        h1 = pltpu.async_copy(scores2_hbm.at[idx_v.at[pl.ds(0, 128)]],
                              rows_v.at[pl.ds(0, 128)], sem)
        h2 = pltpu.async_copy(scores2_hbm.at[idx_v.at[pl.ds(128, 128)]],
                              rows_v.at[pl.ds(128, 128)], sem)
        h1.wait()
        h2.wait()
        pltpu.sync_copy(rows_v, cand_hbm.at[pl.ds(r0 * TOPK, RPW * TOPK)])

    return k2(scores2, gsel.reshape(B * TOPK))


def _k3_body(cand_ref, gsel_ref, ti_ref, w_ref):
    Cd = cand_ref[...]
    gsel = gsel_ref[...]
    base = jnp.concatenate(
        [jnp.broadcast_to(gsel[:, j:j + 1], (B, G)) for j in range(TOPK)],
        axis=1)
    off = lax.rem(lax.broadcasted_iota(jnp.int32, (B, TOPK * G), 1),
                  jnp.int32(G))
    fid = base * jnp.int32(G) + off
    kcol = lax.broadcasted_iota(jnp.int32, (B, TOPK), 1)

    def step(k, carry):
        Cd, ts, fp = carry
        m = jnp.max(Cd, axis=1, keepdims=True)
        sel = jnp.min(jnp.where(Cd == m, fid, BIGI), axis=1, keepdims=True)
        ts = jnp.where(kcol == k, m, ts)
        fp = jnp.where(kcol == k, sel, fp)
        Cd = jnp.where(fid == sel, -jnp.inf, Cd)
        return Cd, ts, fp

    _, ts, fp = lax.fori_loop(
        0, TOPK, step,
        (Cd, jnp.zeros((B, TOPK), jnp.float32), jnp.zeros((B, TOPK), jnp.int32)))
    ti_ref[...] = fp
    w_ref[...] = jax.nn.softmax(ts, axis=-1)


def _k3(cand, gsel):
    return pl.pallas_call(
        _k3_body,
        in_specs=[pl.BlockSpec((B, TOPK * G), lambda: (0, 0)),
                  pl.BlockSpec((B, TOPK), lambda: (0, 0))],
        out_specs=[
            pl.BlockSpec((B, TOPK), lambda: (0, 0)),
            pl.BlockSpec((B, TOPK), lambda: (0, 0)),
        ],
        out_shape=[
            jax.ShapeDtypeStruct((B, TOPK), jnp.int32),
            jax.ShapeDtypeStruct((B, TOPK), jnp.float32),
        ],
    )(cand, gsel)


def _k4(gallery, top_idx):
    mesh = plsc.VectorSubcoreMesh(core_axis_name="c", subcore_axis_name="s")

    @functools.partial(
        pl.kernel,
        mesh=mesh,
        out_type=jax.ShapeDtypeStruct((B, TOPK, SIG), jnp.float32),
        scratch_types=[
            pltpu.VMEM((RPW, TOPK), jnp.int32),
            pltpu.VMEM((2, TOPK, SIG), jnp.float32),
            pltpu.SemaphoreType.DMA,
            pltpu.SemaphoreType.DMA,
        ],
    )
    def k4(gal_hbm, ti_hbm, kv_hbm, idx_v, buf_v, gsem, ssem):
        wid = lax.axis_index("s") * 2 + lax.axis_index("c")
        r0 = wid * RPW
        pltpu.sync_copy(ti_hbm.at[pl.ds(r0, RPW)], idx_v)
        gh = [None] * RPW
        sh = [None] * RPW
        gh[0] = pltpu.async_copy(gal_hbm.at[idx_v.at[0]], buf_v.at[0], gsem)
        for j in range(RPW):
            gh[j].wait()
            if j + 1 < RPW:
                if j >= 1:
                    sh[j - 1].wait()
                gh[j + 1] = pltpu.async_copy(
                    gal_hbm.at[idx_v.at[j + 1]], buf_v.at[(j + 1) % 2], gsem)
            sh[j] = pltpu.async_copy(buf_v.at[j % 2], kv_hbm.at[r0 + j], ssem)
        sh[RPW - 2].wait()
        sh[RPW - 1].wait()

    return k4(gallery, top_idx)


K5_R = 1024


def _k5_body(kv_ref, wb_ref, b_ref, gma_ref, bta_ref, wgt_ref, out_ref):
    x = kv_ref[...]
    s2 = jnp.sum(x * x, axis=1, keepdims=True)
    n = jnp.maximum(jnp.sqrt(s2), 1e-12)
    gb = (x / n).astype(jnp.bfloat16)
    t = lax.dot_general(gb, wb_ref[...], (((1,), (0,)), ((), ())),
                        preferred_element_type=jnp.float32)
    t = t + b_ref[...]
    mu = jnp.mean(t, axis=1, keepdims=True)
    var = jnp.mean((t - mu) ** 2, axis=1, keepdims=True)
    y = (t - mu) / jnp.sqrt(var + 1e-5) * gma_ref[...] + bta_ref[...]
    out_ref[...] = y * wgt_ref[...]


def _k5(kv2, w_bf, b2, gma2, bta2, wgt2):
    nstep = (B * TOPK) // K5_R
    return pl.pallas_call(
        _k5_body,
        grid=(nstep,),
        in_specs=[
            pl.BlockSpec((K5_R, SIG), lambda i: (i, 0)),
            pl.BlockSpec((SIG, D), lambda i: (0, 0)),
            pl.BlockSpec((1, D), lambda i: (0, 0)),
            pl.BlockSpec((1, D), lambda i: (0, 0)),
            pl.BlockSpec((1, D), lambda i: (0, 0)),
            pl.BlockSpec((K5_R, 1), lambda i: (i, 0)),
        ],
        out_specs=pl.BlockSpec((K5_R, D), lambda i: (i, 0)),
        out_shape=jax.ShapeDtypeStruct((B * TOPK, D), jnp.float32),
    )(kv2, w_bf, b2, gma2, bta2, wgt2)


def kernel(emb_a, emb_b, gallery_sigs, W_sig, b_sig, ln_gamma, ln_beta):
    q_bf = _pair_signature(emb_a, emb_b).astype(jnp.bfloat16)
    scores, gmax = _k1(q_bf, gallery_sigs)
    gsel = _k15(gmax)
    cand = _k2(scores.reshape(B * NG, G), gsel)
    top_idx, wgt = _k3(cand.reshape(B, TOPK * G), gsel)
    kv = _k4(gallery_sigs, top_idx)
    out = _k5(
        kv.reshape(B * TOPK, SIG),
        W_sig.astype(jnp.bfloat16),
        b_sig.reshape(1, D),
        ln_gamma.reshape(1, D),
        ln_beta.reshape(1, D),
        wgt.reshape(B * TOPK, 1),
    )
    return out.reshape(B, TOPK, D)


# speedup vs baseline: 1.0060x; 1.0060x over previous
import functools

import jax
import jax.numpy as jnp
from jax import lax
from jax.experimental import pallas as pl
from jax.experimental.pallas import tpu as pltpu
from jax.experimental.pallas import tpu_sc as plsc

B = 256
D = 512
SIG = 1536
TOPK = 32
N = 100000
C = 3584
NSTEP = 28
NPAD = NSTEP * C
G = 128
PG = C // G
NG = NSTEP * PG
NW = 32
RPW = B // NW
BIGI = 2**30


def _l2norm(x, eps=1e-12):
    n = jnp.sqrt(jnp.sum(x * x, axis=-1, keepdims=True))
    return x / jnp.maximum(n, eps)


def _pair_signature(emb_a, emb_b):
    a = _l2norm(emb_a)
    b = _l2norm(emb_b)
    sig = jnp.concatenate([a + b, jnp.abs(a - b), a * b], axis=-1)
    return _l2norm(sig)


def _k1_body(qb_ref, gal_ref, sc_ref, gsel_ref, gm_ref):
    i = pl.program_id(0)
    x = gal_ref[...]
    s2 = jnp.sum(x * x, axis=1, keepdims=True)
    n = jnp.maximum(jnp.sqrt(s2), 1e-12)
    gb = (x / n).astype(jnp.bfloat16)
    sc = lax.dot_general(qb_ref[...], gb, (((1,), (1,)), ((), ())),
                         preferred_element_type=jnp.float32)
    valid = jnp.int32(N) - i * jnp.int32(C)
    lane = lax.broadcasted_iota(jnp.int32, (B, C), 1)
    sc = jnp.where(lane < valid, sc, -jnp.inf)
    sc_ref[...] = sc
    gm_ref[i] = jnp.max(sc.reshape(B, PG, G), axis=2)

    @pl.when(i == NSTEP - 1)
    def _select():
        M = jnp.concatenate([gm_ref[j] for j in range(NSTEP)], axis=1)
        gid = lax.broadcasted_iota(jnp.int32, (B, NG), 1)
        kcol = lax.broadcasted_iota(jnp.int32, (B, TOPK), 1)

        def sel_step(k, carry):
            M, acc = carry
            m = jnp.max(M, axis=1, keepdims=True)
            sel = jnp.min(jnp.where(M == m, gid, BIGI), axis=1, keepdims=True)
            acc = jnp.where(kcol == k, sel, acc)
            M = jnp.where(gid == sel, -jnp.inf, M)
            return M, acc

        _, S = lax.fori_loop(0, TOPK, sel_step,
                             (M, jnp.zeros((B, TOPK), jnp.int32)))
        gsel_ref[...] = S


def _k1(q_bf, gallery):
    return pl.pallas_call(
        _k1_body,
        grid=(NSTEP,),
        in_specs=[
            pl.BlockSpec((B, SIG), lambda i: (0, 0)),
            pl.BlockSpec((C, SIG), lambda i: (i, 0)),
        ],
        out_specs=[
            pl.BlockSpec((B, C), lambda i: (0, i)),
            pl.BlockSpec((B, TOPK), lambda i: (0, 0)),
        ],
        out_shape=[
            jax.ShapeDtypeStruct((B, NPAD), jnp.float32),
            jax.ShapeDtypeStruct((B, TOPK), jnp.int32),
        ],
        scratch_shapes=[pltpu.VMEM((NSTEP, B, PG), jnp.float32)],
    )(q_bf, gallery)


def _k2(scores2, gsel):
    mesh = plsc.VectorSubcoreMesh(core_axis_name="c", subcore_axis_name="s")

    @functools.partial(
        pl.kernel,
        mesh=mesh,
        out_type=jax.ShapeDtypeStruct((B * TOPK, G), jnp.float32),
        scratch_types=[
            pltpu.VMEM((RPW * TOPK,), jnp.int32),
            pltpu.VMEM((RPW * TOPK, G), jnp.float32),
            pltpu.SemaphoreType.DMA,
        ],
    )
    def k2(scores2_hbm, gsel2_hbm, cand_hbm, idx_v, rows_v, sem):
        wid = lax.axis_index("s") * 2 + lax.axis_index("c")
        r0 = wid * RPW
        pltpu.sync_copy(gsel2_hbm.at[pl.ds(r0 * TOPK, RPW * TOPK)], idx_v)
        for j in range(RPW):
            base = (r0 + j) * jnp.int32(NG)
            for h in (0, 16):
                p = j * TOPK + h
                idx_v[pl.ds(p, 16)] = idx_v[pl.ds(p, 16)] + base
        h1 = pltpu.async_copy(scores2_hbm.at[idx_v.at[pl.ds(0, 128)]],
                              rows_v.at[pl.ds(0, 128)], sem)
        h2 = pltpu.async_copy(scores2_hbm.at[idx_v.at[pl.ds(128, 128)]],
                              rows_v.at[pl.ds(128, 128)], sem)
        h1.wait()
        h2.wait()
        pltpu.sync_copy(rows_v, cand_hbm.at[pl.ds(r0 * TOPK, RPW * TOPK)])

    return k2(scores2, gsel.reshape(B * TOPK))


def _k3_body(cand_ref, gsel_ref, ti_ref, w_ref):
    Cd = cand_ref[...]
    gsel = gsel_ref[...]
    base = jnp.concatenate(
        [jnp.broadcast_to(gsel[:, j:j + 1], (B, G)) for j in range(TOPK)],
        axis=1)
    off = lax.rem(lax.broadcasted_iota(jnp.int32, (B, TOPK * G), 1),
                  jnp.int32(G))
    fid = base * jnp.int32(G) + off
    kcol = lax.broadcasted_iota(jnp.int32, (B, TOPK), 1)

    def step(k, carry):
        Cd, ts, fp = carry
        m = jnp.max(Cd, axis=1, keepdims=True)
        sel = jnp.min(jnp.where(Cd == m, fid, BIGI), axis=1, keepdims=True)
        ts = jnp.where(kcol == k, m, ts)
        fp = jnp.where(kcol == k, sel, fp)
        Cd = jnp.where(fid == sel, -jnp.inf, Cd)
        return Cd, ts, fp

    _, ts, fp = lax.fori_loop(
        0, TOPK, step,
        (Cd, jnp.zeros((B, TOPK), jnp.float32), jnp.zeros((B, TOPK), jnp.int32)))
    ti_ref[...] = fp
    w_ref[...] = jax.nn.softmax(ts, axis=-1)


def _k3(cand, gsel):
    return pl.pallas_call(
        _k3_body,
        in_specs=[pl.BlockSpec((B, TOPK * G), lambda: (0, 0)),
                  pl.BlockSpec((B, TOPK), lambda: (0, 0))],
        out_specs=[
            pl.BlockSpec((B, TOPK), lambda: (0, 0)),
            pl.BlockSpec((B, TOPK), lambda: (0, 0)),
        ],
        out_shape=[
            jax.ShapeDtypeStruct((B, TOPK), jnp.int32),
            jax.ShapeDtypeStruct((B, TOPK), jnp.float32),
        ],
    )(cand, gsel)


def _k4(gallery, top_idx):
    mesh = plsc.VectorSubcoreMesh(core_axis_name="c", subcore_axis_name="s")

    @functools.partial(
        pl.kernel,
        mesh=mesh,
        out_type=jax.ShapeDtypeStruct((B, TOPK, SIG), jnp.float32),
        scratch_types=[
            pltpu.VMEM((RPW, TOPK), jnp.int32),
            pltpu.VMEM((2, TOPK, SIG), jnp.float32),
            pltpu.SemaphoreType.DMA,
            pltpu.SemaphoreType.DMA,
        ],
    )
    def k4(gal_hbm, ti_hbm, kv_hbm, idx_v, buf_v, gsem, ssem):
        wid = lax.axis_index("s") * 2 + lax.axis_index("c")
        r0 = wid * RPW
        pltpu.sync_copy(ti_hbm.at[pl.ds(r0, RPW)], idx_v)
        gh = [None] * RPW
        sh = [None] * RPW
        gh[0] = pltpu.async_copy(gal_hbm.at[idx_v.at[0]], buf_v.at[0], gsem)
        for j in range(RPW):
            gh[j].wait()
            if j + 1 < RPW:
                if j >= 1:
                    sh[j - 1].wait()
                gh[j + 1] = pltpu.async_copy(
                    gal_hbm.at[idx_v.at[j + 1]], buf_v.at[(j + 1) % 2], gsem)
            sh[j] = pltpu.async_copy(buf_v.at[j % 2], kv_hbm.at[r0 + j], ssem)
        sh[RPW - 2].wait()
        sh[RPW - 1].wait()

    return k4(gallery, top_idx)


K5_R = 1024


def _k5_body(kv_ref, wb_ref, b_ref, gma_ref, bta_ref, wgt_ref, out_ref):
    x = kv_ref[...]
    s2 = jnp.sum(x * x, axis=1, keepdims=True)
    n = jnp.maximum(jnp.sqrt(s2), 1e-12)
    gb = (x / n).astype(jnp.bfloat16)
    t = lax.dot_general(gb, wb_ref[...], (((1,), (0,)), ((), ())),
                        preferred_element_type=jnp.float32)
    t = t + b_ref[...]
    mu = jnp.mean(t, axis=1, keepdims=True)
    var = jnp.mean((t - mu) ** 2, axis=1, keepdims=True)
    y = (t - mu) / jnp.sqrt(var + 1e-5) * gma_ref[...] + bta_ref[...]
    out_ref[...] = y * wgt_ref[...]


def _k5(kv2, w_bf, b2, gma2, bta2, wgt2):
    nstep = (B * TOPK) // K5_R
    return pl.pallas_call(
        _k5_body,
        grid=(nstep,),
        in_specs=[
            pl.BlockSpec((K5_R, SIG), lambda i: (i, 0)),
            pl.BlockSpec((SIG, D), lambda i: (0, 0)),
            pl.BlockSpec((1, D), lambda i: (0, 0)),
            pl.BlockSpec((1, D), lambda i: (0, 0)),
            pl.BlockSpec((1, D), lambda i: (0, 0)),
            pl.BlockSpec((K5_R, 1), lambda i: (i, 0)),
        ],
        out_specs=pl.BlockSpec((K5_R, D), lambda i: (i, 0)),
        out_shape=jax.ShapeDtypeStruct((B * TOPK, D), jnp.float32),
    )(kv2, w_bf, b2, gma2, bta2, wgt2)


def kernel(emb_a, emb_b, gallery_sigs, W_sig, b_sig, ln_gamma, ln_beta):
    q_bf = _pair_signature(emb_a, emb_b).astype(jnp.bfloat16)
    scores, gsel = _k1(q_bf, gallery_sigs)
    cand = _k2(scores.reshape(B * NG, G), gsel)
    top_idx, wgt = _k3(cand.reshape(B, TOPK * G), gsel)
    kv = _k4(gallery_sigs, top_idx)
    out = _k5(
        kv.reshape(B * TOPK, SIG),
        W_sig.astype(jnp.bfloat16),
        b_sig.reshape(1, D),
        ln_gamma.reshape(1, D),
        ln_beta.reshape(1, D),
        wgt.reshape(B * TOPK, 1),
    )
    return out.reshape(B, TOPK, D)
